# baseline (speedup 1.0000x reference)
import jax
import jax.numpy as jnp
from jax.experimental import pallas as pl

_CHUNK = 2048
_BB = 1


def _add_kernel(x_ref, p_ref, o_ref):
    o_ref[...] = x_ref[...] + p_ref[...][None, :, :]


def kernel(inputs, pos_table):
    b, s, d = inputs.shape
    chunk = min(_CHUNK, s)
    bb = min(_BB, b)
    return pl.pallas_call(
        _add_kernel,
        grid=(s // chunk, b // bb),
        in_specs=[
            pl.BlockSpec((bb, chunk, d), lambda i, j: (j, i, 0)),
            pl.BlockSpec((chunk, d), lambda i, j: (i, 0)),
        ],
        out_specs=pl.BlockSpec((bb, chunk, d), lambda i, j: (j, i, 0)),
        out_shape=jax.ShapeDtypeStruct((b, s, d), inputs.dtype),
    )(inputs, pos_table)
